# initial kernel scaffold (unmeasured)
import jax
import jax.numpy as jnp
from jax import lax
from jax.experimental import pallas as pl
from jax.experimental.pallas import tpu as pltpu

N_DEV = 4


def kernel(x, w_mat, scale_x, scale_w):
    m_per, k = x.shape
    _, n = w_mat.shape
    n_per = n // N_DEV

    my = lax.axis_index("i")
    w_my = lax.dynamic_slice_in_dim(w_mat, my * n_per, n_per, axis=1)
    w_bf = w_my.astype(jnp.bfloat16)
    s = (scale_x * scale_w).astype(jnp.float32).reshape(1, 1)

    def body(x_ref, w_ref, s_ref, out_ref,
             comm_ref, blk_ref, send_sems, recv_sems, copy_sem):
        my_pos = lax.axis_index("i")
        left = (my_pos + N_DEV - 1) % N_DEV
        right = (my_pos + 1) % N_DEV

        barrier_sem = pltpu.get_barrier_semaphore()
        for nbr in (left, right):
            pl.semaphore_signal(
                barrier_sem, inc=1,
                device_id=(nbr,), device_id_type=pl.DeviceIdType.MESH,
            )
        pl.semaphore_wait(barrier_sem, 2)

        comm_ref[0] = x_ref[...]
        scale = s_ref[0, 0]

        def compute_block(slot, origin):
            xb = comm_ref[slot].astype(jnp.bfloat16)
            acc = jnp.dot(xb, w_ref[...], preferred_element_type=jnp.float32)
            y = acc * scale
            blk_ref[...] = y * jax.nn.sigmoid(y)
            cp = pltpu.make_async_copy(
                blk_ref,
                out_ref.at[pl.ds(origin * m_per, m_per), :],
                copy_sem,
            )
            cp.start()
            cp.wait()

        for h in range(N_DEV - 1):
            rdma = pltpu.make_async_remote_copy(
                src_ref=comm_ref.at[h],
                dst_ref=comm_ref.at[h + 1],
                send_sem=send_sems.at[h],
                recv_sem=recv_sems.at[h],
                device_id=(right,),
                device_id_type=pl.DeviceIdType.MESH,
            )
            rdma.start()
            compute_block(h, (my_pos - h) % N_DEV)
            rdma.wait()
        compute_block(N_DEV - 1, (my_pos - (N_DEV - 1)) % N_DEV)

    return pl.pallas_call(
        body,
        out_shape=jax.ShapeDtypeStruct((N_DEV * m_per, n_per), jnp.float32),
        in_specs=[
            pl.BlockSpec(memory_space=pltpu.VMEM),
            pl.BlockSpec(memory_space=pltpu.VMEM),
            pl.BlockSpec(memory_space=pltpu.SMEM),
        ],
        out_specs=pl.BlockSpec(memory_space=pltpu.ANY),
        scratch_shapes=[
            pltpu.VMEM((N_DEV, m_per, k), jnp.int8),
            pltpu.VMEM((m_per, n_per), jnp.float32),
            pltpu.SemaphoreType.DMA((N_DEV - 1,)),
            pltpu.SemaphoreType.DMA((N_DEV - 1,)),
            pltpu.SemaphoreType.DMA,
        ],
        compiler_params=pltpu.CompilerParams(collective_id=0),
    )(x, w_bf, s)


# baseline (device time: 210557 ns/iter reference)
import jax
import jax.numpy as jnp
from jax import lax
from jax.experimental import pallas as pl
from jax.experimental.pallas import tpu as pltpu

N_DEV = 4


def kernel(x, w_mat, scale_x, scale_w):
    m_per, k = x.shape
    _, n = w_mat.shape
    n_per = n // N_DEV

    my = lax.axis_index("i")
    w_my = lax.dynamic_slice_in_dim(w_mat, my * n_per, n_per, axis=1)
    w_bf = w_my.astype(jnp.bfloat16)
    s = (scale_x * scale_w).astype(jnp.float32).reshape(1, 1)

    def body(x_ref, w_hbm_ref, s_ref, out_ref,
             comm_ref, w_ref, blk_ref, send_sems, recv_sems, copy_sem,
             w_sem):
        my_pos = lax.axis_index("i")
        left = (my_pos + N_DEV - 1) % N_DEV
        right = (my_pos + 1) % N_DEV

        barrier_sem = pltpu.get_barrier_semaphore()
        for nbr in (left, right):
            pl.semaphore_signal(
                barrier_sem, inc=1,
                device_id=(nbr,), device_id_type=pl.DeviceIdType.MESH,
            )
        pl.semaphore_wait(barrier_sem, 2)

        cp_x = pltpu.make_async_copy(x_ref, comm_ref.at[0], copy_sem)
        cp_w = pltpu.make_async_copy(w_hbm_ref, w_ref, w_sem)
        cp_x.start()
        cp_w.start()
        cp_x.wait()
        cp_w.wait()
        scale = s_ref[0, 0]

        n_half = n_per // 2

        def compute_block(slot, origin):
            xb = comm_ref[slot].astype(jnp.bfloat16)
            for j in range(2):
                sl = slice(j * n_half, (j + 1) * n_half)
                acc = jnp.dot(
                    xb, w_ref[:, sl], preferred_element_type=jnp.float32
                )
                y = acc * scale
                blk_ref[:, sl] = y * jax.nn.sigmoid(y)
            cp = pltpu.make_async_copy(
                blk_ref,
                out_ref.at[pl.ds(origin * m_per, m_per), :],
                copy_sem,
            )
            cp.start()
            cp.wait()

        for h in range(N_DEV - 1):
            rdma = pltpu.make_async_remote_copy(
                src_ref=comm_ref.at[h],
                dst_ref=comm_ref.at[h + 1],
                send_sem=send_sems.at[h],
                recv_sem=recv_sems.at[h],
                device_id=(right,),
                device_id_type=pl.DeviceIdType.MESH,
            )
            rdma.start()
            compute_block(h, (my_pos - h) % N_DEV)
            rdma.wait()
        compute_block(N_DEV - 1, (my_pos - (N_DEV - 1)) % N_DEV)

    return pl.pallas_call(
        body,
        out_shape=jax.ShapeDtypeStruct((N_DEV * m_per, n_per), jnp.float32),
        in_specs=[
            pl.BlockSpec(memory_space=pl.ANY),
            pl.BlockSpec(memory_space=pl.ANY),
            pl.BlockSpec(memory_space=pltpu.SMEM),
        ],
        out_specs=pl.BlockSpec(memory_space=pl.ANY),
        scratch_shapes=[
            pltpu.VMEM((N_DEV, m_per, k), jnp.int8),
            pltpu.VMEM((k, n_per), jnp.bfloat16),
            pltpu.VMEM((m_per, n_per), jnp.float32),
            pltpu.SemaphoreType.DMA((N_DEV - 1,)),
            pltpu.SemaphoreType.DMA((N_DEV - 1,)),
            pltpu.SemaphoreType.DMA,
            pltpu.SemaphoreType.DMA,
        ],
        compiler_params=pltpu.CompilerParams(
            collective_id=0,
            vmem_limit_bytes=58 * 1024 * 1024,
        ),
    )(x, w_bf, s)
